# baseline (speedup 1.0000x reference)
import jax
import jax.numpy as jnp
from jax.experimental import pallas as pl
from jax.experimental.pallas import tpu as pltpu

_R, _C = 1_000_000, 64
_NCH = 125
_CR = _R // _NCH
_K = 8
_L = 4


def _in_copy(a_ref, buf_ref, insem, chunk, slot):
    return pltpu.make_async_copy(
        a_ref.at[pl.ds(chunk * _CR, _CR)], buf_ref.at[slot], insem.at[slot]
    )


def _out_copy(o_ref, buf_ref, outsem, chunk, slot):
    return pltpu.make_async_copy(
        buf_ref.at[slot], o_ref.at[pl.ds(chunk * _CR, _CR)], outsem.at[slot]
    )


def _body(a_ref, o_ref, buf_ref, fix_ref, insem, outsem, fsem_in, fsem_out):
    for k in range(_L):
        _in_copy(a_ref, buf_ref, insem, k, k).start()

    def step(i, carry):
        slot = jax.lax.rem(i, _K)
        _in_copy(a_ref, buf_ref, insem, i, slot).wait()
        _out_copy(o_ref, buf_ref, outsem, i, slot).start()

        j = i + _L
        jslot = jax.lax.rem(j, _K)

        @pl.when(j < _NCH)
        def _start_next():
            @pl.when(j >= _K)
            def _free_buffer():
                _out_copy(o_ref, buf_ref, outsem, j - _K, jslot).wait()

            _in_copy(a_ref, buf_ref, insem, j, jslot).start()

        return carry

    jax.lax.fori_loop(0, _NCH, step, 0)

    for k in range(_K):
        chunk = _NCH - _K + k
        _out_copy(o_ref, buf_ref, outsem, chunk, chunk % _K).wait()

    cin = pltpu.make_async_copy(a_ref.at[pl.ds(0, 8)], fix_ref, fsem_in)
    cin.start()
    cin.wait()
    r = jax.lax.broadcasted_iota(jnp.int32, (8, _C), 0)
    c = jax.lax.broadcasted_iota(jnp.int32, (8, _C), 1)
    upd = jnp.where((r == 0) & (c == 0), 1.0, 0.0) + jnp.where(
        (r == 1) & (c == 0), 2.0, 0.0
    )
    fix_ref[...] += upd.astype(fix_ref.dtype)
    cout = pltpu.make_async_copy(fix_ref, o_ref.at[pl.ds(0, 8)], fsem_out)
    cout.start()
    cout.wait()


def kernel(A):
    return pl.pallas_call(
        _body,
        in_specs=[pl.BlockSpec(memory_space=pl.ANY)],
        out_specs=pl.BlockSpec(memory_space=pl.ANY),
        out_shape=jax.ShapeDtypeStruct((_R, _C), A.dtype),
        scratch_shapes=[
            pltpu.VMEM((_K, _CR, _C), A.dtype),
            pltpu.VMEM((8, _C), A.dtype),
            pltpu.SemaphoreType.DMA((_K,)),
            pltpu.SemaphoreType.DMA((_K,)),
            pltpu.SemaphoreType.DMA,
            pltpu.SemaphoreType.DMA,
        ],
    )(A)


# speedup vs baseline: 3.1319x; 3.1319x over previous
import jax
import jax.numpy as jnp
from jax.experimental import pallas as pl
from jax.experimental.pallas import tpu as pltpu

_R, _C = 1_000_000, 64
_H = 16


def _scatter_body(a_ref, o_ref):
    r = jax.lax.broadcasted_iota(jnp.int32, (_H, _C), 0)
    c = jax.lax.broadcasted_iota(jnp.int32, (_H, _C), 1)
    upd = jnp.where((r == 0) & (c == 0), 1.0, 0.0) + jnp.where(
        (r == 1) & (c == 0), 2.0, 0.0
    )
    o_ref[...] = a_ref[...] + upd.astype(o_ref.dtype)


def kernel(A):
    head = pl.pallas_call(
        _scatter_body,
        out_shape=jax.ShapeDtypeStruct((_H, _C), A.dtype),
    )(A[:_H])
    return jnp.concatenate([head, A[_H:]], axis=0)


# speedup vs baseline: 6.1815x; 1.9737x over previous
import jax
import jax.numpy as jnp
from jax.experimental import pallas as pl
from jax.experimental.pallas import tpu as pltpu

_R, _C = 1_000_000, 64
_H = 16


def _scatter_body(a_ref, o_ref):
    r = jax.lax.broadcasted_iota(jnp.int32, (_H, _C), 0)
    c = jax.lax.broadcasted_iota(jnp.int32, (_H, _C), 1)
    upd = jnp.where((r == 0) & (c == 0), 1.0, 0.0) + jnp.where(
        (r == 1) & (c == 0), 2.0, 0.0
    )
    o_ref[...] = a_ref[...] + upd.astype(o_ref.dtype)


def kernel(A):
    head = pl.pallas_call(
        _scatter_body,
        out_shape=jax.ShapeDtypeStruct((_H, _C), A.dtype),
    )(A[:_H])
    return jax.lax.dynamic_update_slice(A, head, (0, 0))
